# baseline (device time: 30266 ns/iter reference)
import jax
import jax.numpy as jnp
from jax import lax
from jax.experimental import pallas as pl
from jax.experimental.pallas import tpu as pltpu

N_DEV = 16


def kernel(x, w_mat, scale_x, scale_w):
    m_global, k_local = x.shape
    k_global, n = w_mat.shape
    m_per = m_global // N_DEV

    def body(x_ref, w_ref, sx_ref, sw_ref, out_ref, xfull_ref,
             send_sems, recv_sems):
        my = lax.axis_index("i")

        sends = []
        for j in range(1, N_DEV):
            tgt = lax.rem(my + j, N_DEV)
            rdma = pltpu.make_async_remote_copy(
                src_ref=x_ref.at[pl.ds(tgt * m_per, m_per), :],
                dst_ref=xfull_ref.at[:, pl.ds(my * k_local, k_local)],
                send_sem=send_sems.at[j],
                recv_sem=recv_sems.at[my],
                device_id=(tgt,),
                device_id_type=pl.DeviceIdType.MESH,
            )
            rdma.start()
            sends.append(rdma)

        xfull_ref[:, pl.ds(my * k_local, k_local)] = x_ref[
            pl.ds(my * m_per, m_per), :
        ]

        for j in range(1, N_DEV):
            src = lax.rem(my - j + N_DEV, N_DEV)
            recv = pltpu.make_async_remote_copy(
                src_ref=x_ref.at[pl.ds(0, m_per), :],
                dst_ref=xfull_ref.at[:, pl.ds(src * k_local, k_local)],
                send_sem=send_sems.at[0],
                recv_sem=recv_sems.at[src],
                device_id=(src,),
                device_id_type=pl.DeviceIdType.MESH,
            )
            recv.wait_recv()

        acc = jnp.dot(
            xfull_ref[...], w_ref[...], preferred_element_type=jnp.int32
        )
        out_ref[...] = acc.astype(jnp.float32) * (sx_ref[0] * sw_ref[0])

        for rdma in sends:
            rdma.wait_send()

    return pl.pallas_call(
        body,
        out_shape=jax.ShapeDtypeStruct((m_per, n), jnp.float32),
        in_specs=[
            pl.BlockSpec(memory_space=pltpu.VMEM),
            pl.BlockSpec(memory_space=pltpu.VMEM),
            pl.BlockSpec(memory_space=pltpu.SMEM),
            pl.BlockSpec(memory_space=pltpu.SMEM),
        ],
        out_specs=pl.BlockSpec(memory_space=pltpu.VMEM),
        scratch_shapes=[
            pltpu.VMEM((m_per, k_global), jnp.int8),
            pltpu.SemaphoreType.DMA((N_DEV,)),
            pltpu.SemaphoreType.DMA((N_DEV,)),
        ],
    )(x, w_mat, scale_x, scale_w)


# device time: 26779 ns/iter; 1.1302x vs baseline; 1.1302x over previous
import jax
import jax.numpy as jnp
from jax import lax
from jax.experimental import pallas as pl
from jax.experimental.pallas import tpu as pltpu

N_DEV = 16


def kernel(x, w_mat, scale_x, scale_w):
    m_global, k_local = x.shape
    k_global, n = w_mat.shape
    m_per = m_global // N_DEV

    def body(x_ref, w_ref, sx_ref, sw_ref, out_ref, xfull_ref,
             send_sems, recv_sems):
        my = lax.axis_index("i")

        barrier_sem = pltpu.get_barrier_semaphore()
        for j in range(1, N_DEV):
            pl.semaphore_signal(
                barrier_sem, inc=1,
                device_id=(lax.rem(my + j, N_DEV),),
                device_id_type=pl.DeviceIdType.MESH,
            )
        pl.semaphore_wait(barrier_sem, N_DEV - 1)

        sends = []
        for j in range(1, N_DEV):
            tgt = lax.rem(my + j, N_DEV)
            rdma = pltpu.make_async_remote_copy(
                src_ref=x_ref.at[pl.ds(tgt * m_per, m_per), :],
                dst_ref=xfull_ref.at[:, pl.ds(my * k_local, k_local)],
                send_sem=send_sems.at[j],
                recv_sem=recv_sems.at[my],
                device_id=(tgt,),
                device_id_type=pl.DeviceIdType.MESH,
            )
            rdma.start()
            sends.append(rdma)

        xfull_ref[:, pl.ds(my * k_local, k_local)] = x_ref[
            pl.ds(my * m_per, m_per), :
        ]

        for j in range(1, N_DEV):
            src = lax.rem(my - j + N_DEV, N_DEV)
            recv = pltpu.make_async_remote_copy(
                src_ref=x_ref.at[pl.ds(0, m_per), :],
                dst_ref=xfull_ref.at[:, pl.ds(src * k_local, k_local)],
                send_sem=send_sems.at[0],
                recv_sem=recv_sems.at[src],
                device_id=(src,),
                device_id_type=pl.DeviceIdType.MESH,
            )
            recv.wait_recv()

        acc = jnp.dot(
            xfull_ref[...], w_ref[...], preferred_element_type=jnp.int32
        )
        out_ref[...] = acc.astype(jnp.float32) * (sx_ref[0] * sw_ref[0])

        for rdma in sends:
            rdma.wait_send()

    return pl.pallas_call(
        body,
        out_shape=jax.ShapeDtypeStruct((m_per, n), jnp.float32),
        in_specs=[
            pl.BlockSpec(memory_space=pltpu.VMEM),
            pl.BlockSpec(memory_space=pltpu.VMEM),
            pl.BlockSpec(memory_space=pltpu.SMEM),
            pl.BlockSpec(memory_space=pltpu.SMEM),
        ],
        out_specs=pl.BlockSpec(memory_space=pltpu.VMEM),
        scratch_shapes=[
            pltpu.VMEM((m_per, k_global), jnp.int8),
            pltpu.SemaphoreType.DMA((N_DEV,)),
            pltpu.SemaphoreType.DMA((N_DEV,)),
        ],
        compiler_params=pltpu.CompilerParams(collective_id=0),
    )(x, w_mat, scale_x, scale_w)


# device time: 21976 ns/iter; 1.3772x vs baseline; 1.2186x over previous
import jax
import jax.numpy as jnp
from jax import lax
from jax.experimental import pallas as pl
from jax.experimental.pallas import tpu as pltpu

N_DEV = 16
N_GROUPS = 4


def kernel(x, w_mat, scale_x, scale_w):
    m_global, k_local = x.shape
    k_global, n = w_mat.shape
    m_per = m_global // N_DEV

    def body(x_ref, w_ref, sx_ref, sw_ref, out_ref, xfull_ref, acc_ref,
             send_sems, recv_sems):
        my = lax.axis_index("i")

        barrier_sem = pltpu.get_barrier_semaphore()
        pl.semaphore_signal(
            barrier_sem, inc=1, device_id=(my,),
            device_id_type=pl.DeviceIdType.MESH,
        )
        pl.semaphore_wait(barrier_sem, 1)

        sends = []
        for j in range(1, N_DEV):
            tgt = lax.rem(my + j, N_DEV)
            rdma = pltpu.make_async_remote_copy(
                src_ref=x_ref.at[pl.ds(tgt * m_per, m_per), :],
                dst_ref=xfull_ref.at[:, pl.ds(my * k_local, k_local)],
                send_sem=send_sems.at[j],
                recv_sem=recv_sems.at[my],
                device_id=(tgt,),
                device_id_type=pl.DeviceIdType.MESH,
            )
            rdma.start()
            sends.append(rdma)

        xfull_ref[:, pl.ds(my * k_local, k_local)] = x_ref[
            pl.ds(my * m_per, m_per), :
        ]

        gk = k_global // N_GROUPS
        per_g = N_DEV // N_GROUPS
        for g in range(N_GROUPS):
            for p in range(per_g * g, per_g * (g + 1)):
                recv = pltpu.make_async_remote_copy(
                    src_ref=x_ref.at[pl.ds(0, m_per), :],
                    dst_ref=xfull_ref.at[:, pl.ds(p * k_local, k_local)],
                    send_sem=send_sems.at[0],
                    recv_sem=recv_sems.at[p],
                    device_id=(p,),
                    device_id_type=pl.DeviceIdType.MESH,
                )

                @pl.when(p != my)
                def _(recv=recv):
                    recv.wait_recv()

            blk = jnp.dot(
                xfull_ref[:, pl.ds(g * gk, gk)],
                w_ref[pl.ds(g * gk, gk), :],
                preferred_element_type=jnp.int32,
            )
            if g == 0:
                acc_ref[...] = blk
            else:
                acc_ref[...] += blk

        out_ref[...] = acc_ref[...].astype(jnp.float32) * (
            sx_ref[0] * sw_ref[0]
        )

        for rdma in sends:
            rdma.wait_send()

    return pl.pallas_call(
        body,
        out_shape=jax.ShapeDtypeStruct((m_per, n), jnp.float32),
        in_specs=[
            pl.BlockSpec(memory_space=pltpu.VMEM),
            pl.BlockSpec(memory_space=pltpu.VMEM),
            pl.BlockSpec(memory_space=pltpu.SMEM),
            pl.BlockSpec(memory_space=pltpu.SMEM),
        ],
        out_specs=pl.BlockSpec(memory_space=pltpu.VMEM),
        scratch_shapes=[
            pltpu.VMEM((m_per, k_global), jnp.int8),
            pltpu.VMEM((m_per, n), jnp.int32),
            pltpu.SemaphoreType.DMA((N_DEV,)),
            pltpu.SemaphoreType.DMA((N_DEV,)),
        ],
        compiler_params=pltpu.CompilerParams(collective_id=0),
    )(x, w_mat, scale_x, scale_w)
